# baseline (speedup 1.0000x reference)
import functools

import jax
import jax.numpy as jnp
import numpy as np
from jax import lax
from jax.experimental import pallas as pl
from jax.experimental.pallas import tpu as pltpu
from jax.experimental.pallas import tpu_sc as plsc

N = 10000
E = 320000
NG = 10
NPG = 1000
FP = 48
CH = 80
RPT = 256
SB = 32
EPT = 20000
NP = 10240
RT = NP // 16

INV_BN = float(1.0 / np.sqrt(1.0 + 1e-5))


def _dot(a, b):
    return jnp.dot(a, b, preferred_element_type=jnp.float32)


def _leaky(x):
    return jnp.where(x >= 0, x, 0.01 * x)


def _seg_body(fp, nbuf, pa, pb, srca, dsta, ewa, srcb, dstb, ewb, out,
              *scr):
    acc, src_v, dst_v, ew_v = scr[:4]
    rows = scr[4:4 + nbuf]
    gsem = scr[4 + nbuf:4 + 2 * nbuf]
    ssem = scr[4 + 2 * nbuf:4 + 3 * nbuf]
    c = lax.axis_index("c")
    s = lax.axis_index("s")

    def zrow(j, _):
        for t in range(fp // 16):
            rows[0][j, pl.ds(16 * t, 16)] = jnp.zeros((16,), jnp.float32)
        return 0
    lax.fori_loop(0, CH, zrow, 0)
    for k in range(RT // CH):
        pltpu.sync_copy(rows[0], acc.at[pl.ds(s * RT + k * CH, CH)])
    plsc.subcore_barrier()

    def process(p_hbm, src3, dst3, ew3):
        def drain(buf, sem):
            pltpu.make_async_copy(p_hbm.at[pl.ds(0, CH)], buf, sem).wait()

        def scale(buf, i):
            def grp(g, _):
                wv = ew_v[i, pl.ds(g * 16, 16)]
                base = g * 16
                for t in range(16):
                    w = wv[t]
                    for tt in range(fp // 16):
                        sl = pl.ds(16 * tt, 16)
                        buf[base + t, sl] = buf[base + t, sl] * w
                return 0
            lax.fori_loop(0, CH // 16, grp, 0)

        def half(j, i):
            drain(rows[j], gsem[j])
            scale(rows[j], i)
            pltpu.async_copy(rows[j], acc.at[dst_v.at[i]], ssem[j], add=True)

        def rearm(m, i_next):
            drain(rows[m], ssem[m])
            pltpu.async_copy(p_hbm.at[src_v.at[i_next]], rows[m], gsem[m])

        def block(b, _):
            pltpu.sync_copy(src3.at[s].at[pl.ds(b * SB, SB)], src_v)
            pltpu.sync_copy(dst3.at[s].at[pl.ds(b * SB, SB)], dst_v)
            pltpu.sync_copy(ew3.at[s].at[pl.ds(b * SB, SB)], ew_v)
            for j in range(nbuf):
                pltpu.async_copy(p_hbm.at[src_v.at[j]], rows[j], gsem[j])

            if nbuf == 2:
                def pair(k, __):
                    i0 = 2 * k
                    half(0, i0)
                    half(1, i0 + 1)
                    rearm(0, i0 + 2)
                    rearm(1, i0 + 3)
                    return 0
                lax.fori_loop(0, SB // 2 - 1, pair, 0)
                half(0, SB - 2)
                half(1, SB - 1)
            else:
                for j in range(4):
                    half(j, j)
                    if j >= 2:
                        rearm((j + 2) % 4, j + 2)

                def quad(k, __):
                    i0 = 4 * k
                    for j in range(4):
                        half(j, i0 + j)
                        rearm((j + 2) % 4, i0 + j + 2)
                    return 0
                lax.fori_loop(1, SB // 4 - 1, quad, 0)
                i0 = SB - 4
                for j in range(4):
                    half(j, i0 + j)
                    if j < 2:
                        rearm((j + 2) % 4, i0 + j + 2)
            for j in range(nbuf):
                drain(rows[j], ssem[j])
            return 0
        lax.fori_loop(0, RPT // SB, block, 0)

    @pl.when(c == 0)
    def _a():
        process(pa, srca, dsta, ewa)

    @pl.when(c == 1)
    def _b():
        process(pb, srcb, dstb, ewb)

    plsc.subcore_barrier()
    row0 = s * RT
    pltpu.sync_copy(acc.at[pl.ds(row0, RT)], out.at[c, pl.ds(row0, RT)])


@functools.lru_cache(maxsize=None)
def _make_seg(fp, nbuf):
    return functools.partial(
        pl.kernel,
        out_type=jax.ShapeDtypeStruct((2, NP, fp), jnp.float32),
        mesh=plsc.VectorSubcoreMesh(core_axis_name="c", subcore_axis_name="s"),
        compiler_params=pltpu.CompilerParams(use_tc_tiling_on_sc=False),
        scratch_types=[
            pltpu.VMEM_SHARED((NP, fp), jnp.float32),
            pltpu.VMEM((SB, CH), jnp.int32),
            pltpu.VMEM((SB, CH), jnp.int32),
            pltpu.VMEM((SB, CH), jnp.float32),
        ] + [pltpu.VMEM((CH, fp), jnp.float32)] * nbuf
          + [pltpu.SemaphoreType.DMA] * (2 * nbuf),
    )(functools.partial(_seg_body, fp, nbuf))


def _seg(fp, xa, xb, slabs):
    srca, dsta, ewa, srcb, dstb, ewb = slabs
    nbuf = 2 if fp == 128 else 4
    return _make_seg(fp, nbuf)(xa, xb, srca, dsta, ewa, srcb, dstb,
                               ewb)[:, :N, :]


def _mask48(h):
    lane = lax.broadcasted_iota(jnp.int32, h.shape, 1)
    return jnp.where(lane < 40, h, 0.0)


def _gconv_tail(agg, x, wr_ref, wo_ref, b_ref, gs_ref, gb_ref):
    g = _dot(agg, wr_ref[...]) + _dot(x, wo_ref[...]) + b_ref[0, :][None, :]
    h = _leaky(g)
    h = (h * INV_BN) * gs_ref[0, :][:, None] + gb_ref[0, :][:, None]
    return _mask48(h)


def _tc_mid_body(s_ref, xa_ref, xb_ref, wra_ref, wrb_ref, woa_ref, wob_ref,
                 ba_ref, bb_ref, gsa_ref, gba_ref, gsb_ref, gbb_ref,
                 ha_ref, hb_ref):
    s = s_ref[...]
    ha_ref[...] = _gconv_tail(s[0], xa_ref[...], wra_ref, woa_ref, ba_ref,
                              gsa_ref, gba_ref)
    hb_ref[...] = _gconv_tail(s[1], xb_ref[...], wrb_ref, wob_ref, bb_ref,
                              gsb_ref, gbb_ref)


def _tcf_body(s3_ref, ha_ref, hb_ref, wra_ref, wrb_ref, woa_ref, wob_ref,
              ba_ref, bb_ref, fa_w_ref, fa_b_ref, fb_w_ref, fb_b_ref,
              f2_w_ref, f2_b_ref, out_ref):
    s3 = s3_ref[...]
    outs = []
    for chi, h_ref, wr_ref, wo_ref, b_ref, fw_ref, fb_ref in (
            (0, ha_ref, wra_ref, woa_ref, ba_ref, fa_w_ref, fa_b_ref),
            (1, hb_ref, wrb_ref, wob_ref, bb_ref, fb_w_ref, fb_b_ref)):
        z = (_dot(s3[chi], wr_ref[...]) + _dot(h_ref[...], wo_ref[...])
             + b_ref[0, :][None, :])
        h3 = _leaky(z[:, 0:1]).reshape(NG, NPG)
        m = jnp.mean(h3, axis=1, keepdims=True)
        cen = h3 - m
        v = jnp.sum(cen * cen, axis=1, keepdims=True) * (1.0 / (NPG - 1))
        hn = cen / (v + 1e-10)
        o = lax.dot_general(hn, fw_ref[...], (((1,), (1,)), ((), ())),
                            preferred_element_type=jnp.float32)
        outs.append(o + fb_ref[0, :][None, :])
    u = _leaky(jnp.concatenate(outs, axis=1))
    out_ref[...] = lax.dot_general(u, f2_w_ref[...], (((1,), (1,)), ((), ())),
                                   preferred_element_type=jnp.float32) \
        + f2_b_ref[0, :][None, :]


def _row_spec(w):
    return pl.BlockSpec((NPG, w), lambda i: (i, 0))


def _full_spec(shape):
    return pl.BlockSpec(shape, lambda i: tuple(0 for _ in shape))


def _tc_mid(s, xa, xb, wra, wrb, woa, wob, ba, bb, gsa, gba, gsb, gbb, win):
    return pl.pallas_call(
        _tc_mid_body,
        grid=(NG,),
        in_specs=[pl.BlockSpec((2, NPG, win), lambda i: (0, i, 0)),
                  _row_spec(win), _row_spec(win),
                  _full_spec((win, FP)), _full_spec((win, FP)),
                  _full_spec((win, FP)), _full_spec((win, FP)),
                  _full_spec((1, FP)), _full_spec((1, FP)),
                  _full_spec((1, NPG)), _full_spec((1, NPG)),
                  _full_spec((1, NPG)), _full_spec((1, NPG))],
        out_specs=[_row_spec(FP)] * 2,
        out_shape=[jax.ShapeDtypeStruct((N, FP), jnp.float32)] * 2,
    )(s, xa, xb, wra, wrb, woa, wob, ba, bb, gsa, gba, gsb, gbb)


def _tcf(s3, ha, hb, wra, wrb, woa, wob, ba, bb, fa_w, fa_b, fb_w, fb_b,
         f2_w, f2_b):
    return pl.pallas_call(
        _tcf_body,
        out_shape=jax.ShapeDtypeStruct((NG, NG), jnp.float32),
    )(s3, ha, hb, wra, wrb, woa, wob, ba, bb, fa_w, fa_b, fb_w, fb_b,
      f2_w, f2_b)


def _pad_t(w, fin, fout):
    return jnp.zeros((fin, fout), jnp.float32).at[:w.shape[1], :w.shape[0]].set(w.T)


def _pad_b(b, fout):
    return jnp.zeros((1, fout), jnp.float32).at[0, :b.shape[0]].set(b)


def _slab(a, fill):
    a2 = a.reshape(16, EPT)
    pad = jnp.full((16, RPT * CH - EPT), fill, a.dtype)
    return jnp.concatenate([a2, pad], axis=1).reshape(16, RPT, CH)


def kernel(x, edge_index, edge_attr, feature_node, feature_edge_index, features, num_graphs, c11_rel, c11_root, c11_bias, c12_rel, c12_root, c12_bias, c13_rel, c13_root, c13_bias, bn11_g, bn11_b, bn12_g, bn12_b, fc11_w, fc11_b, c21_rel, c21_root, c21_bias, c22_rel, c22_root, c22_bias, c23_rel, c23_root, c23_bias, bn21_g, bn21_b, bn22_g, bn22_b, fc21_w, fc21_b, fc2_w, fc2_b):
    slabs = (_slab(edge_index[0], 0), _slab(edge_index[1], N),
             _slab(edge_attr, 0.0),
             _slab(feature_edge_index[0], 0), _slab(feature_edge_index[1], N),
             _slab(features, 0.0))

    s1 = _seg(128, x, feature_node, slabs)
    h1a, h1b = _tc_mid(
        s1, x, feature_node,
        _pad_t(c11_rel, 128, FP), _pad_t(c21_rel, 128, FP),
        _pad_t(c11_root, 128, FP), _pad_t(c21_root, 128, FP),
        _pad_b(c11_bias, FP), _pad_b(c21_bias, FP),
        bn11_g.reshape(1, NPG), bn11_b.reshape(1, NPG),
        bn21_g.reshape(1, NPG), bn21_b.reshape(1, NPG), 128)

    s2 = _seg(FP, h1a, h1b, slabs)
    h2a, h2b = _tc_mid(
        s2, h1a, h1b,
        _pad_t(c12_rel, FP, FP), _pad_t(c22_rel, FP, FP),
        _pad_t(c12_root, FP, FP), _pad_t(c22_root, FP, FP),
        _pad_b(c12_bias, FP), _pad_b(c22_bias, FP),
        bn12_g.reshape(1, NPG), bn12_b.reshape(1, NPG),
        bn22_g.reshape(1, NPG), bn22_b.reshape(1, NPG), FP)

    s3 = _seg(FP, h2a, h2b, slabs)
    out = _tcf(
        s3, h2a, h2b,
        jnp.tile(_pad_t(c13_rel, FP, 1), (1, 16)),
        jnp.tile(_pad_t(c23_rel, FP, 1), (1, 16)),
        jnp.tile(_pad_t(c13_root, FP, 1), (1, 16)),
        jnp.tile(_pad_t(c23_root, FP, 1), (1, 16)),
        jnp.tile(_pad_b(c13_bias, 1), (1, 16)),
        jnp.tile(_pad_b(c23_bias, 1), (1, 16)),
        fc11_w, fc11_b.reshape(1, -1),
        fc21_w, fc21_b.reshape(1, -1),
        fc2_w, fc2_b.reshape(1, -1))
    return out + (jnp.asarray(num_graphs) - NG).astype(out.dtype)


# speedup vs baseline: 1.5796x; 1.5796x over previous
import functools

import jax
import jax.numpy as jnp
import numpy as np
from jax import lax
from jax.experimental import pallas as pl
from jax.experimental.pallas import tpu as pltpu
from jax.experimental.pallas import tpu_sc as plsc

N = 10000
E = 320000
NG = 10
NPG = 1000
FP = 48
CH = 80
RPT = 250
SB = 50
EPT = 20000
NP = 10240
RT = NP // 16

INV_BN = float(1.0 / np.sqrt(1.0 + 1e-5))


def _dot(a, b):
    return jnp.dot(a, b, preferred_element_type=jnp.float32)


def _leaky(x):
    return jnp.where(x >= 0, x, 0.01 * x)


def _seg_body(fp, pa, pb, srca, dsta, ewa, srcb, dstb, ewb, out,
              acc, src_v, dst_v, ew_v, rows_a, rows_b,
              sem_ga, sem_gb, sem_sa, sem_sb):
    c = lax.axis_index("c")
    s = lax.axis_index("s")

    def zrow(j, _):
        for t in range(fp // 16):
            rows_a[j, pl.ds(16 * t, 16)] = jnp.zeros((16,), jnp.float32)
        return 0
    lax.fori_loop(0, CH, zrow, 0)
    for k in range(RT // CH):
        pltpu.sync_copy(rows_a, acc.at[pl.ds(s * RT + k * CH, CH)])
    plsc.subcore_barrier()

    def process(p_hbm, src3, dst3, ew3):
        def drain(buf, sem):
            pltpu.make_async_copy(p_hbm.at[pl.ds(0, CH)], buf, sem).wait()

        def scale(buf, i):
            def grp(g, _):
                wv = ew_v[i, pl.ds(g * 16, 16)]
                base = g * 16
                for t in range(16):
                    w = wv[t]
                    for tt in range(fp // 16):
                        sl = pl.ds(16 * tt, 16)
                        buf[base + t, sl] = buf[base + t, sl] * w
                return 0
            lax.fori_loop(0, CH // 16, grp, 0)

        def half(buf, gs, ss, i):
            drain(buf, gs)
            scale(buf, i)
            pltpu.async_copy(buf, acc.at[dst_v.at[i]], ss, add=True)

        def block(b, _):
            pltpu.sync_copy(src3.at[s].at[pl.ds(b * SB, SB)], src_v)
            pltpu.sync_copy(dst3.at[s].at[pl.ds(b * SB, SB)], dst_v)
            pltpu.sync_copy(ew3.at[s].at[pl.ds(b * SB, SB)], ew_v)
            pltpu.async_copy(p_hbm.at[src_v.at[0]], rows_a, sem_ga)
            pltpu.async_copy(p_hbm.at[src_v.at[1]], rows_b, sem_gb)

            def pair(k, __):
                i0 = 2 * k
                half(rows_a, sem_ga, sem_sa, i0)
                half(rows_b, sem_gb, sem_sb, i0 + 1)
                drain(rows_a, sem_sa)
                pltpu.async_copy(p_hbm.at[src_v.at[i0 + 2]], rows_a, sem_ga)
                drain(rows_b, sem_sb)
                pltpu.async_copy(p_hbm.at[src_v.at[i0 + 3]], rows_b, sem_gb)
                return 0
            lax.fori_loop(0, SB // 2 - 1, pair, 0)
            half(rows_a, sem_ga, sem_sa, SB - 2)
            half(rows_b, sem_gb, sem_sb, SB - 1)
            drain(rows_a, sem_sa)
            drain(rows_b, sem_sb)
            return 0
        lax.fori_loop(0, RPT // SB, block, 0)

    @pl.when(c == 0)
    def _a():
        process(pa, srca, dsta, ewa)

    @pl.when(c == 1)
    def _b():
        process(pb, srcb, dstb, ewb)

    plsc.subcore_barrier()
    row0 = s * RT
    pltpu.sync_copy(acc.at[pl.ds(row0, RT)], out.at[c, pl.ds(row0, RT)])


@functools.lru_cache(maxsize=None)
def _make_seg(fp):
    return functools.partial(
        pl.kernel,
        out_type=jax.ShapeDtypeStruct((2, NP, fp), jnp.float32),
        mesh=plsc.VectorSubcoreMesh(core_axis_name="c", subcore_axis_name="s"),
        compiler_params=pltpu.CompilerParams(use_tc_tiling_on_sc=False),
        scratch_types=[
            pltpu.VMEM_SHARED((NP, fp), jnp.float32),
            pltpu.VMEM((SB, CH), jnp.int32),
            pltpu.VMEM((SB, CH), jnp.int32),
            pltpu.VMEM((SB, CH), jnp.float32),
            pltpu.VMEM((CH, fp), jnp.float32),
            pltpu.VMEM((CH, fp), jnp.float32),
            pltpu.SemaphoreType.DMA,
            pltpu.SemaphoreType.DMA,
            pltpu.SemaphoreType.DMA,
            pltpu.SemaphoreType.DMA,
        ],
    )(functools.partial(_seg_body, fp))


def _seg(fp, xa, xb, slabs):
    srca, dsta, ewa, srcb, dstb, ewb = slabs
    return _make_seg(fp)(xa, xb, srca, dsta, ewa, srcb, dstb, ewb)[:, :N, :]


def _mask48(h):
    lane = lax.broadcasted_iota(jnp.int32, h.shape, 1)
    return jnp.where(lane < 40, h, 0.0)


def _gconv_tail(agg, x, wr_ref, wo_ref, b_ref, gs_ref, gb_ref):
    g = _dot(agg, wr_ref[...]) + _dot(x, wo_ref[...]) + b_ref[0, :][None, :]
    h = _leaky(g)
    h = (h * INV_BN) * gs_ref[0, :][:, None] + gb_ref[0, :][:, None]
    return _mask48(h)


def _tc_mid_body(s_ref, xa_ref, xb_ref, wra_ref, wrb_ref, woa_ref, wob_ref,
                 ba_ref, bb_ref, gsa_ref, gba_ref, gsb_ref, gbb_ref,
                 ha_ref, hb_ref):
    s = s_ref[...]
    ha_ref[...] = _gconv_tail(s[0], xa_ref[...], wra_ref, woa_ref, ba_ref,
                              gsa_ref, gba_ref)
    hb_ref[...] = _gconv_tail(s[1], xb_ref[...], wrb_ref, wob_ref, bb_ref,
                              gsb_ref, gbb_ref)


def _tcf_body(s3_ref, ha_ref, hb_ref, wra_ref, wrb_ref, woa_ref, wob_ref,
              ba_ref, bb_ref, fa_w_ref, fa_b_ref, fb_w_ref, fb_b_ref,
              f2_w_ref, f2_b_ref, out_ref):
    s3 = s3_ref[...]
    outs = []
    for chi, h_ref, wr_ref, wo_ref, b_ref, fw_ref, fb_ref in (
            (0, ha_ref, wra_ref, woa_ref, ba_ref, fa_w_ref, fa_b_ref),
            (1, hb_ref, wrb_ref, wob_ref, bb_ref, fb_w_ref, fb_b_ref)):
        z = (_dot(s3[chi], wr_ref[...]) + _dot(h_ref[...], wo_ref[...])
             + b_ref[0, :][None, :])
        h3 = _leaky(z[:, 0:1]).reshape(NG, NPG)
        m = jnp.mean(h3, axis=1, keepdims=True)
        cen = h3 - m
        v = jnp.sum(cen * cen, axis=1, keepdims=True) * (1.0 / (NPG - 1))
        hn = cen / (v + 1e-10)
        o = lax.dot_general(hn, fw_ref[...], (((1,), (1,)), ((), ())),
                            preferred_element_type=jnp.float32)
        outs.append(o + fb_ref[0, :][None, :])
    u = _leaky(jnp.concatenate(outs, axis=1))
    out_ref[...] = lax.dot_general(u, f2_w_ref[...], (((1,), (1,)), ((), ())),
                                   preferred_element_type=jnp.float32) \
        + f2_b_ref[0, :][None, :]


def _row_spec(w):
    return pl.BlockSpec((NPG, w), lambda i: (i, 0))


def _full_spec(shape):
    return pl.BlockSpec(shape, lambda i: tuple(0 for _ in shape))


def _tc_mid(s, xa, xb, wra, wrb, woa, wob, ba, bb, gsa, gba, gsb, gbb, win):
    return pl.pallas_call(
        _tc_mid_body,
        grid=(NG,),
        in_specs=[pl.BlockSpec((2, NPG, win), lambda i: (0, i, 0)),
                  _row_spec(win), _row_spec(win),
                  _full_spec((win, FP)), _full_spec((win, FP)),
                  _full_spec((win, FP)), _full_spec((win, FP)),
                  _full_spec((1, FP)), _full_spec((1, FP)),
                  _full_spec((1, NPG)), _full_spec((1, NPG)),
                  _full_spec((1, NPG)), _full_spec((1, NPG))],
        out_specs=[_row_spec(FP)] * 2,
        out_shape=[jax.ShapeDtypeStruct((N, FP), jnp.float32)] * 2,
    )(s, xa, xb, wra, wrb, woa, wob, ba, bb, gsa, gba, gsb, gbb)


def _tcf(s3, ha, hb, wra, wrb, woa, wob, ba, bb, fa_w, fa_b, fb_w, fb_b,
         f2_w, f2_b):
    return pl.pallas_call(
        _tcf_body,
        out_shape=jax.ShapeDtypeStruct((NG, NG), jnp.float32),
    )(s3, ha, hb, wra, wrb, woa, wob, ba, bb, fa_w, fa_b, fb_w, fb_b,
      f2_w, f2_b)


def _pad_t(w, fin, fout):
    return jnp.zeros((fin, fout), jnp.float32).at[:w.shape[1], :w.shape[0]].set(w.T)


def _pad_b(b, fout):
    return jnp.zeros((1, fout), jnp.float32).at[0, :b.shape[0]].set(b)


def _slab(a, fill):
    del fill
    return a.reshape(16, RPT, CH)


def kernel(x, edge_index, edge_attr, feature_node, feature_edge_index, features, num_graphs, c11_rel, c11_root, c11_bias, c12_rel, c12_root, c12_bias, c13_rel, c13_root, c13_bias, bn11_g, bn11_b, bn12_g, bn12_b, fc11_w, fc11_b, c21_rel, c21_root, c21_bias, c22_rel, c22_root, c22_bias, c23_rel, c23_root, c23_bias, bn21_g, bn21_b, bn22_g, bn22_b, fc21_w, fc21_b, fc2_w, fc2_b):
    slabs = (_slab(edge_index[0], 0), _slab(edge_index[1], N),
             _slab(edge_attr, 0.0),
             _slab(feature_edge_index[0], 0), _slab(feature_edge_index[1], N),
             _slab(features, 0.0))

    s1 = _seg(128, x, feature_node, slabs)
    h1a, h1b = _tc_mid(
        s1, x, feature_node,
        _pad_t(c11_rel, 128, FP), _pad_t(c21_rel, 128, FP),
        _pad_t(c11_root, 128, FP), _pad_t(c21_root, 128, FP),
        _pad_b(c11_bias, FP), _pad_b(c21_bias, FP),
        bn11_g.reshape(1, NPG), bn11_b.reshape(1, NPG),
        bn21_g.reshape(1, NPG), bn21_b.reshape(1, NPG), 128)

    s2 = _seg(FP, h1a, h1b, slabs)
    h2a, h2b = _tc_mid(
        s2, h1a, h1b,
        _pad_t(c12_rel, FP, FP), _pad_t(c22_rel, FP, FP),
        _pad_t(c12_root, FP, FP), _pad_t(c22_root, FP, FP),
        _pad_b(c12_bias, FP), _pad_b(c22_bias, FP),
        bn12_g.reshape(1, NPG), bn12_b.reshape(1, NPG),
        bn22_g.reshape(1, NPG), bn22_b.reshape(1, NPG), FP)

    s3 = _seg(FP, h2a, h2b, slabs)
    out = _tcf(
        s3, h2a, h2b,
        jnp.tile(_pad_t(c13_rel, FP, 1), (1, 16)),
        jnp.tile(_pad_t(c23_rel, FP, 1), (1, 16)),
        jnp.tile(_pad_t(c13_root, FP, 1), (1, 16)),
        jnp.tile(_pad_t(c23_root, FP, 1), (1, 16)),
        jnp.tile(_pad_b(c13_bias, 1), (1, 16)),
        jnp.tile(_pad_b(c23_bias, 1), (1, 16)),
        fc11_w, fc11_b.reshape(1, -1),
        fc21_w, fc21_b.reshape(1, -1),
        fc2_w, fc2_b.reshape(1, -1))
    return out + (jnp.asarray(num_graphs) - NG).astype(out.dtype)


# speedup vs baseline: 1.6069x; 1.0173x over previous
import functools

import jax
import jax.numpy as jnp
import numpy as np
from jax import lax
from jax.experimental import pallas as pl
from jax.experimental.pallas import tpu as pltpu
from jax.experimental.pallas import tpu_sc as plsc

N = 10000
E = 320000
NG = 10
NPG = 1000
FP = 48
CH = 80
RPT = 250
SB = 50
SB48 = 125
EPT = 20000
NP = 10240
RT = NP // 16

INV_BN = float(1.0 / np.sqrt(1.0 + 1e-5))


def _dot(a, b):
    return jnp.dot(a, b, preferred_element_type=jnp.float32)


def _leaky(x):
    return jnp.where(x >= 0, x, 0.01 * x)


def _seg_body(fp, sb, pa, pb, srca, dsta, ewa, srcb, dstb, ewb, out,
              acc, src_v, dst_v, ew_v, rows_a, rows_b,
              sem_ga, sem_gb, sem_sa, sem_sb):
    c = lax.axis_index("c")
    s = lax.axis_index("s")

    def zrow(j, _):
        for t in range(fp // 16):
            rows_a[j, pl.ds(16 * t, 16)] = jnp.zeros((16,), jnp.float32)
        return 0
    lax.fori_loop(0, CH, zrow, 0)
    for k in range(RT // CH):
        pltpu.sync_copy(rows_a, acc.at[pl.ds(s * RT + k * CH, CH)])
    plsc.subcore_barrier()

    def process(p_hbm, src3, dst3, ew3):
        def drain(buf, sem):
            pltpu.make_async_copy(p_hbm.at[pl.ds(0, CH)], buf, sem).wait()

        def scale(buf, i):
            def grp(g, _):
                wv = ew_v[i, pl.ds(g * 16, 16)]
                base = g * 16
                for t in range(16):
                    w = wv[t]
                    for tt in range(fp // 16):
                        sl = pl.ds(16 * tt, 16)
                        buf[base + t, sl] = buf[base + t, sl] * w
                return 0
            lax.fori_loop(0, CH // 16, grp, 0)

        def half(buf, gs, ss, i):
            drain(buf, gs)
            scale(buf, i)
            pltpu.async_copy(buf, acc.at[dst_v.at[i]], ss, add=True)

        def block(b, _):
            pltpu.sync_copy(src3.at[s].at[pl.ds(b * sb, sb)], src_v)
            pltpu.sync_copy(dst3.at[s].at[pl.ds(b * sb, sb)], dst_v)
            pltpu.sync_copy(ew3.at[s].at[pl.ds(b * sb, sb)], ew_v)
            pltpu.async_copy(p_hbm.at[src_v.at[0]], rows_a, sem_ga)
            pltpu.async_copy(p_hbm.at[src_v.at[1]], rows_b, sem_gb)

            def pair(k, __):
                i0 = 2 * k
                half(rows_a, sem_ga, sem_sa, i0)
                half(rows_b, sem_gb, sem_sb, i0 + 1)
                drain(rows_a, sem_sa)
                pltpu.async_copy(p_hbm.at[src_v.at[i0 + 2]], rows_a, sem_ga)
                drain(rows_b, sem_sb)
                pltpu.async_copy(p_hbm.at[src_v.at[i0 + 3]], rows_b, sem_gb)
                return 0
            lax.fori_loop(0, sb // 2 - 1, pair, 0)
            half(rows_a, sem_ga, sem_sa, sb - 2)
            half(rows_b, sem_gb, sem_sb, sb - 1)
            drain(rows_a, sem_sa)
            drain(rows_b, sem_sb)
            return 0
        lax.fori_loop(0, RPT // sb, block, 0)

    @pl.when(c == 0)
    def _a():
        process(pa, srca, dsta, ewa)

    @pl.when(c == 1)
    def _b():
        process(pb, srcb, dstb, ewb)

    plsc.subcore_barrier()
    row0 = s * RT
    pltpu.sync_copy(acc.at[pl.ds(row0, RT)], out.at[c, pl.ds(row0, RT)])


@functools.lru_cache(maxsize=None)
def _make_seg(fp):
    sb = SB if fp == 128 else SB48
    return functools.partial(
        pl.kernel,
        out_type=jax.ShapeDtypeStruct((2, NP, fp), jnp.float32),
        mesh=plsc.VectorSubcoreMesh(core_axis_name="c", subcore_axis_name="s"),
        compiler_params=pltpu.CompilerParams(use_tc_tiling_on_sc=False),
        scratch_types=[
            pltpu.VMEM_SHARED((NP, fp), jnp.float32),
            pltpu.VMEM((sb, CH), jnp.int32),
            pltpu.VMEM((sb, CH), jnp.int32),
            pltpu.VMEM((sb, CH), jnp.float32),
            pltpu.VMEM((CH, fp), jnp.float32),
            pltpu.VMEM((CH, fp), jnp.float32),
            pltpu.SemaphoreType.DMA,
            pltpu.SemaphoreType.DMA,
            pltpu.SemaphoreType.DMA,
            pltpu.SemaphoreType.DMA,
        ],
    )(functools.partial(_seg_body, fp, sb))


def _seg(fp, xa, xb, slabs):
    srca, dsta, ewa, srcb, dstb, ewb = slabs
    return _make_seg(fp)(xa, xb, srca, dsta, ewa, srcb, dstb, ewb)[:, :N, :]


def _mask48(h):
    lane = lax.broadcasted_iota(jnp.int32, h.shape, 1)
    return jnp.where(lane < 40, h, 0.0)


def _gconv_tail(agg, x, wr_ref, wo_ref, b_ref, gs_ref, gb_ref):
    g = _dot(agg, wr_ref[...]) + _dot(x, wo_ref[...]) + b_ref[0, :][None, :]
    h = _leaky(g)
    h = (h * INV_BN) * gs_ref[0, :][:, None] + gb_ref[0, :][:, None]
    return _mask48(h)


def _tc_mid_body(s_ref, xa_ref, xb_ref, wra_ref, wrb_ref, woa_ref, wob_ref,
                 ba_ref, bb_ref, gsa_ref, gba_ref, gsb_ref, gbb_ref,
                 ha_ref, hb_ref):
    s = s_ref[...]
    ha_ref[...] = _gconv_tail(s[0], xa_ref[...], wra_ref, woa_ref, ba_ref,
                              gsa_ref, gba_ref)
    hb_ref[...] = _gconv_tail(s[1], xb_ref[...], wrb_ref, wob_ref, bb_ref,
                              gsb_ref, gbb_ref)


def _tcf_body(s3_ref, ha_ref, hb_ref, wra_ref, wrb_ref, woa_ref, wob_ref,
              ba_ref, bb_ref, fa_w_ref, fa_b_ref, fb_w_ref, fb_b_ref,
              f2_w_ref, f2_b_ref, out_ref):
    s3 = s3_ref[...]
    outs = []
    for chi, h_ref, wr_ref, wo_ref, b_ref, fw_ref, fb_ref in (
            (0, ha_ref, wra_ref, woa_ref, ba_ref, fa_w_ref, fa_b_ref),
            (1, hb_ref, wrb_ref, wob_ref, bb_ref, fb_w_ref, fb_b_ref)):
        z = (_dot(s3[chi], wr_ref[...]) + _dot(h_ref[...], wo_ref[...])
             + b_ref[0, :][None, :])
        h3 = _leaky(z[:, 0:1]).reshape(NG, NPG)
        m = jnp.mean(h3, axis=1, keepdims=True)
        cen = h3 - m
        v = jnp.sum(cen * cen, axis=1, keepdims=True) * (1.0 / (NPG - 1))
        hn = cen / (v + 1e-10)
        o = lax.dot_general(hn, fw_ref[...], (((1,), (1,)), ((), ())),
                            preferred_element_type=jnp.float32)
        outs.append(o + fb_ref[0, :][None, :])
    u = _leaky(jnp.concatenate(outs, axis=1))
    out_ref[...] = lax.dot_general(u, f2_w_ref[...], (((1,), (1,)), ((), ())),
                                   preferred_element_type=jnp.float32) \
        + f2_b_ref[0, :][None, :]


def _row_spec(w):
    return pl.BlockSpec((NPG, w), lambda i: (i, 0))


def _full_spec(shape):
    return pl.BlockSpec(shape, lambda i: tuple(0 for _ in shape))


def _tc_mid(s, xa, xb, wra, wrb, woa, wob, ba, bb, gsa, gba, gsb, gbb, win):
    return pl.pallas_call(
        _tc_mid_body,
        grid=(NG,),
        in_specs=[pl.BlockSpec((2, NPG, win), lambda i: (0, i, 0)),
                  _row_spec(win), _row_spec(win),
                  _full_spec((win, FP)), _full_spec((win, FP)),
                  _full_spec((win, FP)), _full_spec((win, FP)),
                  _full_spec((1, FP)), _full_spec((1, FP)),
                  _full_spec((1, NPG)), _full_spec((1, NPG)),
                  _full_spec((1, NPG)), _full_spec((1, NPG))],
        out_specs=[_row_spec(FP)] * 2,
        out_shape=[jax.ShapeDtypeStruct((N, FP), jnp.float32)] * 2,
    )(s, xa, xb, wra, wrb, woa, wob, ba, bb, gsa, gba, gsb, gbb)


def _tcf(s3, ha, hb, wra, wrb, woa, wob, ba, bb, fa_w, fa_b, fb_w, fb_b,
         f2_w, f2_b):
    return pl.pallas_call(
        _tcf_body,
        out_shape=jax.ShapeDtypeStruct((NG, NG), jnp.float32),
    )(s3, ha, hb, wra, wrb, woa, wob, ba, bb, fa_w, fa_b, fb_w, fb_b,
      f2_w, f2_b)


def _pad_t(w, fin, fout):
    return jnp.zeros((fin, fout), jnp.float32).at[:w.shape[1], :w.shape[0]].set(w.T)


def _pad_b(b, fout):
    return jnp.zeros((1, fout), jnp.float32).at[0, :b.shape[0]].set(b)


def _slab(a, fill):
    del fill
    return a.reshape(16, RPT, CH)


def kernel(x, edge_index, edge_attr, feature_node, feature_edge_index, features, num_graphs, c11_rel, c11_root, c11_bias, c12_rel, c12_root, c12_bias, c13_rel, c13_root, c13_bias, bn11_g, bn11_b, bn12_g, bn12_b, fc11_w, fc11_b, c21_rel, c21_root, c21_bias, c22_rel, c22_root, c22_bias, c23_rel, c23_root, c23_bias, bn21_g, bn21_b, bn22_g, bn22_b, fc21_w, fc21_b, fc2_w, fc2_b):
    slabs = (_slab(edge_index[0], 0), _slab(edge_index[1], N),
             _slab(edge_attr, 0.0),
             _slab(feature_edge_index[0], 0), _slab(feature_edge_index[1], N),
             _slab(features, 0.0))

    s1 = _seg(128, x, feature_node, slabs)
    h1a, h1b = _tc_mid(
        s1, x, feature_node,
        _pad_t(c11_rel, 128, FP), _pad_t(c21_rel, 128, FP),
        _pad_t(c11_root, 128, FP), _pad_t(c21_root, 128, FP),
        _pad_b(c11_bias, FP), _pad_b(c21_bias, FP),
        bn11_g.reshape(1, NPG), bn11_b.reshape(1, NPG),
        bn21_g.reshape(1, NPG), bn21_b.reshape(1, NPG), 128)

    s2 = _seg(FP, h1a, h1b, slabs)
    h2a, h2b = _tc_mid(
        s2, h1a, h1b,
        _pad_t(c12_rel, FP, FP), _pad_t(c22_rel, FP, FP),
        _pad_t(c12_root, FP, FP), _pad_t(c22_root, FP, FP),
        _pad_b(c12_bias, FP), _pad_b(c22_bias, FP),
        bn12_g.reshape(1, NPG), bn12_b.reshape(1, NPG),
        bn22_g.reshape(1, NPG), bn22_b.reshape(1, NPG), FP)

    s3 = _seg(FP, h2a, h2b, slabs)
    out = _tcf(
        s3, h2a, h2b,
        jnp.tile(_pad_t(c13_rel, FP, 1), (1, 16)),
        jnp.tile(_pad_t(c23_rel, FP, 1), (1, 16)),
        jnp.tile(_pad_t(c13_root, FP, 1), (1, 16)),
        jnp.tile(_pad_t(c23_root, FP, 1), (1, 16)),
        jnp.tile(_pad_b(c13_bias, 1), (1, 16)),
        jnp.tile(_pad_b(c23_bias, 1), (1, 16)),
        fc11_w, fc11_b.reshape(1, -1),
        fc21_w, fc21_b.reshape(1, -1),
        fc2_w, fc2_b.reshape(1, -1))
    return out + (jnp.asarray(num_graphs) - NG).astype(out.dtype)


# speedup vs baseline: 1.6100x; 1.0019x over previous
import functools

import jax
import jax.numpy as jnp
import numpy as np
from jax import lax
from jax.experimental import pallas as pl
from jax.experimental.pallas import tpu as pltpu
from jax.experimental.pallas import tpu_sc as plsc

N = 10000
E = 320000
NG = 10
NPG = 1000
FP = 48
CH = 80
RPT = 250
SB = 50
SB48 = 250
EPT = 20000
NP = 10240
RT = NP // 16

INV_BN = float(1.0 / np.sqrt(1.0 + 1e-5))


def _dot(a, b):
    return jnp.dot(a, b, preferred_element_type=jnp.float32)


def _leaky(x):
    return jnp.where(x >= 0, x, 0.01 * x)


def _seg_body(fp, sb, pa, pb, srca, dsta, ewa, srcb, dstb, ewb, out,
              acc, src_v, dst_v, ew_v, rows_a, rows_b,
              sem_ga, sem_gb, sem_sa, sem_sb):
    c = lax.axis_index("c")
    s = lax.axis_index("s")

    def zrow(j, _):
        for t in range(fp // 16):
            rows_a[j, pl.ds(16 * t, 16)] = jnp.zeros((16,), jnp.float32)
        return 0
    lax.fori_loop(0, CH, zrow, 0)
    for k in range(RT // CH):
        pltpu.sync_copy(rows_a, acc.at[pl.ds(s * RT + k * CH, CH)])
    plsc.subcore_barrier()

    def process(p_hbm, src3, dst3, ew3):
        def drain(buf, sem):
            pltpu.make_async_copy(p_hbm.at[pl.ds(0, CH)], buf, sem).wait()

        def scale(buf, i):
            def grp(g, _):
                wv = ew_v[i, pl.ds(g * 16, 16)]
                base = g * 16
                for t in range(16):
                    w = wv[t]
                    for tt in range(fp // 16):
                        sl = pl.ds(16 * tt, 16)
                        buf[base + t, sl] = buf[base + t, sl] * w
                return 0
            lax.fori_loop(0, CH // 16, grp, 0)

        def half(buf, gs, ss, i):
            drain(buf, gs)
            scale(buf, i)
            pltpu.async_copy(buf, acc.at[dst_v.at[i]], ss, add=True)

        def block(b, _):
            pltpu.sync_copy(src3.at[s].at[pl.ds(b * sb, sb)], src_v)
            pltpu.sync_copy(dst3.at[s].at[pl.ds(b * sb, sb)], dst_v)
            pltpu.sync_copy(ew3.at[s].at[pl.ds(b * sb, sb)], ew_v)
            pltpu.async_copy(p_hbm.at[src_v.at[0]], rows_a, sem_ga)
            pltpu.async_copy(p_hbm.at[src_v.at[1]], rows_b, sem_gb)

            def pair(k, __):
                i0 = 2 * k
                half(rows_a, sem_ga, sem_sa, i0)
                half(rows_b, sem_gb, sem_sb, i0 + 1)
                drain(rows_a, sem_sa)
                pltpu.async_copy(p_hbm.at[src_v.at[i0 + 2]], rows_a, sem_ga)
                drain(rows_b, sem_sb)
                pltpu.async_copy(p_hbm.at[src_v.at[i0 + 3]], rows_b, sem_gb)
                return 0
            lax.fori_loop(0, sb // 2 - 1, pair, 0)
            half(rows_a, sem_ga, sem_sa, sb - 2)
            half(rows_b, sem_gb, sem_sb, sb - 1)
            drain(rows_a, sem_sa)
            drain(rows_b, sem_sb)
            return 0
        lax.fori_loop(0, RPT // sb, block, 0)

    @pl.when(c == 0)
    def _a():
        process(pa, srca, dsta, ewa)

    @pl.when(c == 1)
    def _b():
        process(pb, srcb, dstb, ewb)

    plsc.subcore_barrier()
    row0 = s * RT
    pltpu.sync_copy(acc.at[pl.ds(row0, RT)], out.at[c, pl.ds(row0, RT)])


@functools.lru_cache(maxsize=None)
def _make_seg(fp):
    sb = SB if fp == 128 else SB48
    return functools.partial(
        pl.kernel,
        out_type=jax.ShapeDtypeStruct((2, NP, fp), jnp.float32),
        mesh=plsc.VectorSubcoreMesh(core_axis_name="c", subcore_axis_name="s"),
        compiler_params=pltpu.CompilerParams(use_tc_tiling_on_sc=False),
        scratch_types=[
            pltpu.VMEM_SHARED((NP, fp), jnp.float32),
            pltpu.VMEM((sb, CH), jnp.int32),
            pltpu.VMEM((sb, CH), jnp.int32),
            pltpu.VMEM((sb, CH), jnp.float32),
            pltpu.VMEM((CH, fp), jnp.float32),
            pltpu.VMEM((CH, fp), jnp.float32),
            pltpu.SemaphoreType.DMA,
            pltpu.SemaphoreType.DMA,
            pltpu.SemaphoreType.DMA,
            pltpu.SemaphoreType.DMA,
        ],
    )(functools.partial(_seg_body, fp, sb))


def _seg(fp, xa, xb, slabs):
    srca, dsta, ewa, srcb, dstb, ewb = slabs
    return _make_seg(fp)(xa, xb, srca, dsta, ewa, srcb, dstb, ewb)[:, :N, :]


def _mask48(h):
    lane = lax.broadcasted_iota(jnp.int32, h.shape, 1)
    return jnp.where(lane < 40, h, 0.0)


def _gconv_tail(agg, x, wr_ref, wo_ref, b_ref, gs_ref, gb_ref):
    g = _dot(agg, wr_ref[...]) + _dot(x, wo_ref[...]) + b_ref[0, :][None, :]
    h = _leaky(g)
    h = (h * INV_BN) * gs_ref[0, :][:, None] + gb_ref[0, :][:, None]
    return _mask48(h)


def _tc_mid_body(s_ref, xa_ref, xb_ref, wra_ref, wrb_ref, woa_ref, wob_ref,
                 ba_ref, bb_ref, gsa_ref, gba_ref, gsb_ref, gbb_ref,
                 ha_ref, hb_ref):
    s = s_ref[...]
    ha_ref[...] = _gconv_tail(s[0], xa_ref[...], wra_ref, woa_ref, ba_ref,
                              gsa_ref, gba_ref)
    hb_ref[...] = _gconv_tail(s[1], xb_ref[...], wrb_ref, wob_ref, bb_ref,
                              gsb_ref, gbb_ref)


def _tcf_body(s3_ref, ha_ref, hb_ref, wra_ref, wrb_ref, woa_ref, wob_ref,
              ba_ref, bb_ref, fa_w_ref, fa_b_ref, fb_w_ref, fb_b_ref,
              f2_w_ref, f2_b_ref, out_ref):
    s3 = s3_ref[...]
    outs = []
    for chi, h_ref, wr_ref, wo_ref, b_ref, fw_ref, fb_ref in (
            (0, ha_ref, wra_ref, woa_ref, ba_ref, fa_w_ref, fa_b_ref),
            (1, hb_ref, wrb_ref, wob_ref, bb_ref, fb_w_ref, fb_b_ref)):
        z = (_dot(s3[chi], wr_ref[...]) + _dot(h_ref[...], wo_ref[...])
             + b_ref[0, :][None, :])
        h3 = _leaky(z[:, 0:1]).reshape(NG, NPG)
        m = jnp.mean(h3, axis=1, keepdims=True)
        cen = h3 - m
        v = jnp.sum(cen * cen, axis=1, keepdims=True) * (1.0 / (NPG - 1))
        hn = cen / (v + 1e-10)
        o = lax.dot_general(hn, fw_ref[...], (((1,), (1,)), ((), ())),
                            preferred_element_type=jnp.float32)
        outs.append(o + fb_ref[0, :][None, :])
    u = _leaky(jnp.concatenate(outs, axis=1))
    out_ref[...] = lax.dot_general(u, f2_w_ref[...], (((1,), (1,)), ((), ())),
                                   preferred_element_type=jnp.float32) \
        + f2_b_ref[0, :][None, :]


def _row_spec(w):
    return pl.BlockSpec((NPG, w), lambda i: (i, 0))


def _full_spec(shape):
    return pl.BlockSpec(shape, lambda i: tuple(0 for _ in shape))


def _tc_mid(s, xa, xb, wra, wrb, woa, wob, ba, bb, gsa, gba, gsb, gbb, win):
    return pl.pallas_call(
        _tc_mid_body,
        grid=(NG,),
        in_specs=[pl.BlockSpec((2, NPG, win), lambda i: (0, i, 0)),
                  _row_spec(win), _row_spec(win),
                  _full_spec((win, FP)), _full_spec((win, FP)),
                  _full_spec((win, FP)), _full_spec((win, FP)),
                  _full_spec((1, FP)), _full_spec((1, FP)),
                  _full_spec((1, NPG)), _full_spec((1, NPG)),
                  _full_spec((1, NPG)), _full_spec((1, NPG))],
        out_specs=[_row_spec(FP)] * 2,
        out_shape=[jax.ShapeDtypeStruct((N, FP), jnp.float32)] * 2,
    )(s, xa, xb, wra, wrb, woa, wob, ba, bb, gsa, gba, gsb, gbb)


def _tcf(s3, ha, hb, wra, wrb, woa, wob, ba, bb, fa_w, fa_b, fb_w, fb_b,
         f2_w, f2_b):
    return pl.pallas_call(
        _tcf_body,
        out_shape=jax.ShapeDtypeStruct((NG, NG), jnp.float32),
    )(s3, ha, hb, wra, wrb, woa, wob, ba, bb, fa_w, fa_b, fb_w, fb_b,
      f2_w, f2_b)


def _pad_t(w, fin, fout):
    return jnp.zeros((fin, fout), jnp.float32).at[:w.shape[1], :w.shape[0]].set(w.T)


def _pad_b(b, fout):
    return jnp.zeros((1, fout), jnp.float32).at[0, :b.shape[0]].set(b)


def _slab(a, fill):
    del fill
    return a.reshape(16, RPT, CH)


def kernel(x, edge_index, edge_attr, feature_node, feature_edge_index, features, num_graphs, c11_rel, c11_root, c11_bias, c12_rel, c12_root, c12_bias, c13_rel, c13_root, c13_bias, bn11_g, bn11_b, bn12_g, bn12_b, fc11_w, fc11_b, c21_rel, c21_root, c21_bias, c22_rel, c22_root, c22_bias, c23_rel, c23_root, c23_bias, bn21_g, bn21_b, bn22_g, bn22_b, fc21_w, fc21_b, fc2_w, fc2_b):
    slabs = (_slab(edge_index[0], 0), _slab(edge_index[1], N),
             _slab(edge_attr, 0.0),
             _slab(feature_edge_index[0], 0), _slab(feature_edge_index[1], N),
             _slab(features, 0.0))

    s1 = _seg(128, x, feature_node, slabs)
    h1a, h1b = _tc_mid(
        s1, x, feature_node,
        _pad_t(c11_rel, 128, FP), _pad_t(c21_rel, 128, FP),
        _pad_t(c11_root, 128, FP), _pad_t(c21_root, 128, FP),
        _pad_b(c11_bias, FP), _pad_b(c21_bias, FP),
        bn11_g.reshape(1, NPG), bn11_b.reshape(1, NPG),
        bn21_g.reshape(1, NPG), bn21_b.reshape(1, NPG), 128)

    s2 = _seg(FP, h1a, h1b, slabs)
    h2a, h2b = _tc_mid(
        s2, h1a, h1b,
        _pad_t(c12_rel, FP, FP), _pad_t(c22_rel, FP, FP),
        _pad_t(c12_root, FP, FP), _pad_t(c22_root, FP, FP),
        _pad_b(c12_bias, FP), _pad_b(c22_bias, FP),
        bn12_g.reshape(1, NPG), bn12_b.reshape(1, NPG),
        bn22_g.reshape(1, NPG), bn22_b.reshape(1, NPG), FP)

    s3 = _seg(FP, h2a, h2b, slabs)
    out = _tcf(
        s3, h2a, h2b,
        jnp.tile(_pad_t(c13_rel, FP, 1), (1, 16)),
        jnp.tile(_pad_t(c23_rel, FP, 1), (1, 16)),
        jnp.tile(_pad_t(c13_root, FP, 1), (1, 16)),
        jnp.tile(_pad_t(c23_root, FP, 1), (1, 16)),
        jnp.tile(_pad_b(c13_bias, 1), (1, 16)),
        jnp.tile(_pad_b(c23_bias, 1), (1, 16)),
        fc11_w, fc11_b.reshape(1, -1),
        fc21_w, fc21_b.reshape(1, -1),
        fc2_w, fc2_b.reshape(1, -1))
    return out + (jnp.asarray(num_graphs) - NG).astype(out.dtype)
